# baseline (speedup 1.0000x reference)
import jax
import jax.numpy as jnp
from jax import lax
from jax.experimental import pallas as pl
from jax.experimental.pallas import tpu as pltpu
from jax.experimental.pallas import tpu_sc as plsc

_B = 16
_L = 2048
_D = 768
_LRAW = _L + 2
_CH = 32
_NCH = _L // _CH
_NBUF = 3
_MASKP = 2064

_mesh = plsc.VectorSubcoreMesh(
    core_axis_name="c", subcore_axis_name="s", num_cores=2, num_subcores=16)


def _sc_body(emb, maskp, posp, zsrc,
             ctx_hbm, asp_hbm, clen_hbm, alen_hbm,
             buf0, buf1, buf2, zbuf, idx0, idx1, idx2, idxp, maskv, posv,
             lenv, sg0, sg1, sg2, ss0, ss1, ss2, sz, sp, spz):
    cidx = lax.axis_index("c")
    sidx = lax.axis_index("s")
    wid = sidx * 2 + cidx
    bufs = [buf0, buf1, buf2]
    idxs = [idx0, idx1, idx2]
    sg = [sg0, sg1, sg2]
    ss = [ss0, ss1, ss2]
    lanes = jnp.arange(16, dtype=jnp.int32)

    rA = wid
    rB = (wid + 31) & 31
    A_is_ctx = (rA & 1) == 0
    bA = rA >> 1
    bB = rB >> 1
    b_ctx = jnp.where(A_is_ctx, bA, bB)
    b_asp = jnp.where(A_is_ctx, bB, bA)

    pltpu.sync_copy(zsrc, zbuf)
    pltpu.sync_copy(posp.at[b_asp], posv)
    pv = posv[...]
    left = pv[0]
    right = pv[1]

    pltpu.sync_copy(maskp.at[b_ctx], maskv)

    def _mstep(i, acc):
        chunk = maskv[pl.ds(i * 16, 16)]
        return acc + jnp.where(chunk != 0, 1, 0).astype(jnp.int32)

    acc = lax.fori_loop(0, _MASKP // 16, _mstep, jnp.zeros((16,), jnp.int32))
    s_count = jnp.sum(acc)

    nv_ctx = jnp.clip(s_count - 2, 0, _L)
    nv_asp = jnp.clip(right - left, 0, _L)

    lenv[...] = jnp.broadcast_to(jnp.where(A_is_ctx, s_count, right - left),
                                 (16,))

    @pl.when(A_is_ctx)
    def _():
        pltpu.sync_copy(lenv, clen_hbm.at[bA])

    @pl.when(jnp.logical_not(A_is_ctx))
    def _():
        pltpu.sync_copy(lenv, alen_hbm.at[bA])

    def write_idx(iref, base, clamp_hi):
        for q in range(_CH // 16):
            v = jnp.minimum(base + q * 16 + lanes, clamp_hi)
            iref[pl.ds(q * 16, 16)] = v

    def nzeros(nv, p):
        nfull = nv // _CH
        zc0 = nfull + jnp.where(nv - nfull * _CH > 0, 1, 0)
        return _NCH // 2 - (zc0 - p + 1) // 2

    def run(out_hbm, b, nv, src0, p):
        nfull = nv // _CH
        m = nv - nfull * _CH
        mine_m = (m > 0) & ((nfull & 1) == p)
        nk = jnp.maximum((nfull - p + 1) // 2, 0)
        zc0 = nfull + jnp.where(m > 0, 1, 0)
        kz0 = (zc0 - p + 1) // 2

        @pl.when(mine_m)
        def _():
            pltpu.make_async_copy(
                zbuf, out_hbm.at[b, pl.ds(pl.multiple_of(nfull * _CH, _CH),
                                          _CH)], spz).start()

        def zfill(k, carry):
            off = pl.multiple_of((p + 2 * (kz0 + k)) * _CH, _CH)
            pltpu.make_async_copy(
                zbuf, out_hbm.at[b, pl.ds(off, _CH)], sz).start()
            return carry

        lax.fori_loop(0, _NCH // 2 - kz0, zfill, 0)

        def g_start(k, j):
            write_idx(idxs[j], src0 + (p + 2 * k) * _CH, _LRAW - 1)
            pltpu.make_async_copy(
                emb.at[b].at[idxs[j]], bufs[j], sg[j]).start()

        def g_wait(j):
            pltpu.make_async_copy(
                emb.at[b].at[idxs[j]], bufs[j], sg[j]).wait()

        def s_start(k, j):
            off = pl.multiple_of((p + 2 * k) * _CH, _CH)
            pltpu.make_async_copy(
                bufs[j], out_hbm.at[b, pl.ds(off, _CH)], ss[j]).start()

        def s_wait(j):
            pltpu.make_async_copy(
                bufs[j], out_hbm.at[b, pl.ds(0, _CH)], ss[j]).wait()

        for j in range(_NBUF):
            @pl.when(j < nk)
            def _(j=j):
                g_start(j, j)

        def ring(it, carry):
            g = it * _NBUF
            for j in range(_NBUF):
                k = g + j

                @pl.when(k < nk)
                def _(k=k, j=j):
                    g_wait(j)
                    s_start(k, j)

                @pl.when(k + _NBUF < nk)
                def _(k=k, j=j):
                    s_wait(j)
                    g_start(k + _NBUF, j)
            return carry

        lax.fori_loop(0, (nk + _NBUF - 1) // _NBUF, ring, 0)
        for j in range(_NBUF):
            @pl.when(j < nk)
            def _(j=j):
                s_wait(j)

        @pl.when(mine_m)
        def _():
            write_idx(idxp, src0 + nfull * _CH, src0 + nv - 1)
            pltpu.make_async_copy(emb.at[b].at[idxp], buf0, sp).start()
            pltpu.make_async_copy(emb.at[b].at[idxp], buf0, sp).wait()
            pltpu.make_async_copy(
                zbuf, out_hbm.at[b, pl.ds(0, _CH)], spz).wait()
            write_idx(idxp, nfull * _CH, nv - 1)
            pltpu.make_async_copy(buf0, out_hbm.at[b].at[idxp], sp).start()
            pltpu.make_async_copy(buf0, out_hbm.at[b].at[idxp], sp).wait()

    @pl.when(A_is_ctx)
    def _():
        run(ctx_hbm, bA, nv_ctx, 1, 0)
        run(asp_hbm, bB, nv_asp, left, 1)

    @pl.when(jnp.logical_not(A_is_ctx))
    def _():
        run(asp_hbm, bA, nv_asp, left, 0)
        run(ctx_hbm, bB, nv_ctx, 1, 1)

    nz_total = (jnp.where(A_is_ctx, nzeros(nv_ctx, 0) + nzeros(nv_asp, 1),
                          nzeros(nv_asp, 0) + nzeros(nv_ctx, 1)))

    def zwait(i, carry):
        pltpu.make_async_copy(
            zbuf, ctx_hbm.at[0, pl.ds(0, _CH)], sz).wait()
        return carry

    lax.fori_loop(0, nz_total, zwait, 0)


@jax.jit
def kernel(ctx_embeddings, text_mask, aspect_positions):
    maskp = jnp.pad(text_mask, ((0, 0), (0, _MASKP - _LRAW)))
    posp = jnp.pad(aspect_positions, ((0, 0), (0, 14)))
    zsrc = jnp.zeros((_CH, _D), jnp.float32)

    sc_call = pl.kernel(
        _sc_body,
        out_type=[
            jax.ShapeDtypeStruct((_B, _L, _D), jnp.float32),
            jax.ShapeDtypeStruct((_B, _L, _D), jnp.float32),
            jax.ShapeDtypeStruct((_B, 16), jnp.int32),
            jax.ShapeDtypeStruct((_B, 16), jnp.int32),
        ],
        mesh=_mesh,
        compiler_params=pltpu.CompilerParams(needs_layout_passes=False),
        scratch_types=[
            pltpu.VMEM((_CH, _D), jnp.float32),
            pltpu.VMEM((_CH, _D), jnp.float32),
            pltpu.VMEM((_CH, _D), jnp.float32),
            pltpu.VMEM((_CH, _D), jnp.float32),
            pltpu.VMEM((_CH,), jnp.int32),
            pltpu.VMEM((_CH,), jnp.int32),
            pltpu.VMEM((_CH,), jnp.int32),
            pltpu.VMEM((_CH,), jnp.int32),
            pltpu.VMEM((_MASKP,), jnp.int32),
            pltpu.VMEM((16,), jnp.int32),
            pltpu.VMEM((16,), jnp.int32),
        ] + [pltpu.SemaphoreType.DMA] * 9,
    )
    ctx, asp, clen, alen = sc_call(ctx_embeddings, maskp, posp, zsrc)
    return (ctx, asp, clen[:, 0], alen[:, 0])


# speedup vs baseline: 1.0662x; 1.0662x over previous
import jax
import jax.numpy as jnp
from jax import lax
from jax.experimental import pallas as pl
from jax.experimental.pallas import tpu as pltpu
from jax.experimental.pallas import tpu_sc as plsc

_B = 16
_L = 2048
_D = 768
_LRAW = _L + 2
_CH = 32
_NCH = _L // _CH
_NBUF = 3

_mesh = plsc.VectorSubcoreMesh(
    core_axis_name="c", subcore_axis_name="s", num_cores=2, num_subcores=16)


def _sc_body(emb, posp, zsrc, asp_hbm, alen_hbm,
             buf0, buf1, buf2, zbuf, idx0, idx1, idx2, idxp, posv,
             lenv, sg0, sg1, sg2, ss0, ss1, ss2, sz, sp, spz):
    cidx = lax.axis_index("c")
    sidx = lax.axis_index("s")
    wid = sidx * 2 + cidx
    b = wid >> 1
    p = wid & 1
    bufs = [buf0, buf1, buf2]
    idxs = [idx0, idx1, idx2]
    sg = [sg0, sg1, sg2]
    ss = [ss0, ss1, ss2]
    lanes = jnp.arange(16, dtype=jnp.int32)

    pltpu.sync_copy(zsrc, zbuf)
    pltpu.sync_copy(posp.at[b], posv)
    pv = posv[...]
    left = pv[0]
    right = pv[1]
    nv = jnp.clip(right - left, 0, _L)
    src0 = left

    lenv[...] = jnp.broadcast_to(right - left, (16,))

    @pl.when(p == 0)
    def _():
        pltpu.sync_copy(lenv, alen_hbm.at[b])

    def write_idx(iref, base, clamp_hi):
        for q in range(_CH // 16):
            v = jnp.minimum(base + q * 16 + lanes, clamp_hi)
            iref[pl.ds(q * 16, 16)] = v

    nfull = nv // _CH
    m = nv - nfull * _CH
    mine_m = (m > 0) & ((nfull & 1) == p)
    nk = jnp.maximum((nfull - p + 1) // 2, 0)
    zc0 = nfull + jnp.where(m > 0, 1, 0)
    kz0 = (zc0 - p + 1) // 2
    nz = _NCH // 2 - kz0

    @pl.when(mine_m)
    def _():
        pltpu.make_async_copy(
            zbuf, asp_hbm.at[b, pl.ds(pl.multiple_of(nfull * _CH, _CH),
                                      _CH)], spz).start()

    def zfill(k, carry):
        off = pl.multiple_of((p + 2 * (kz0 + k)) * _CH, _CH)
        pltpu.make_async_copy(
            zbuf, asp_hbm.at[b, pl.ds(off, _CH)], sz).start()
        return carry

    lax.fori_loop(0, nz, zfill, 0)

    def g_start(k, j):
        write_idx(idxs[j], src0 + (p + 2 * k) * _CH, _LRAW - 1)
        pltpu.make_async_copy(emb.at[b].at[idxs[j]], bufs[j], sg[j]).start()

    def g_wait(j):
        pltpu.make_async_copy(emb.at[b].at[idxs[j]], bufs[j], sg[j]).wait()

    def s_start(k, j):
        off = pl.multiple_of((p + 2 * k) * _CH, _CH)
        pltpu.make_async_copy(
            bufs[j], asp_hbm.at[b, pl.ds(off, _CH)], ss[j]).start()

    def s_wait(j):
        pltpu.make_async_copy(
            bufs[j], asp_hbm.at[b, pl.ds(0, _CH)], ss[j]).wait()

    for j in range(_NBUF):
        @pl.when(j < nk)
        def _(j=j):
            g_start(j, j)

    def ring(it, carry):
        g = it * _NBUF
        for j in range(_NBUF):
            k = g + j

            @pl.when(k < nk)
            def _(k=k, j=j):
                g_wait(j)
                s_start(k, j)

            @pl.when(k + _NBUF < nk)
            def _(k=k, j=j):
                s_wait(j)
                g_start(k + _NBUF, j)
        return carry

    lax.fori_loop(0, (nk + _NBUF - 1) // _NBUF, ring, 0)
    for j in range(_NBUF):
        @pl.when(j < nk)
        def _(j=j):
            s_wait(j)

    @pl.when(mine_m)
    def _():
        write_idx(idxp, src0 + nfull * _CH, src0 + nv - 1)
        pltpu.make_async_copy(emb.at[b].at[idxp], buf0, sp).start()
        pltpu.make_async_copy(emb.at[b].at[idxp], buf0, sp).wait()
        pltpu.make_async_copy(
            zbuf, asp_hbm.at[b, pl.ds(0, _CH)], spz).wait()
        write_idx(idxp, nfull * _CH, nv - 1)
        pltpu.make_async_copy(buf0, asp_hbm.at[b].at[idxp], sp).start()
        pltpu.make_async_copy(buf0, asp_hbm.at[b].at[idxp], sp).wait()

    def zwait(i, carry):
        pltpu.make_async_copy(
            zbuf, asp_hbm.at[b, pl.ds(0, _CH)], sz).wait()
        return carry

    lax.fori_loop(0, nz, zwait, 0)


def _tc_body(mask_ref, emb_ref, ctx_ref, clen_ref):
    bidx = pl.program_id(0)
    s = jnp.sum((mask_ref[0, 0, :] != 0).astype(jnp.int32))
    clen_ref[bidx] = s
    x = emb_ref[0, 1:_L + 1, :]
    pos = lax.broadcasted_iota(jnp.int32, (_L, 1), 0)
    ctx_ref[0] = jnp.where(pos < s - 2, x, 0.0)


@jax.jit
def kernel(ctx_embeddings, text_mask, aspect_positions):
    mask3 = text_mask.reshape(_B, 1, _LRAW)
    posp = jnp.pad(aspect_positions, ((0, 0), (0, 14)))
    zsrc = jnp.zeros((_CH, _D), jnp.float32)

    sc_call = pl.kernel(
        _sc_body,
        out_type=[
            jax.ShapeDtypeStruct((_B, _L, _D), jnp.float32),
            jax.ShapeDtypeStruct((_B, 16), jnp.int32),
        ],
        mesh=_mesh,
        compiler_params=pltpu.CompilerParams(needs_layout_passes=False),
        scratch_types=[
            pltpu.VMEM((_CH, _D), jnp.float32),
            pltpu.VMEM((_CH, _D), jnp.float32),
            pltpu.VMEM((_CH, _D), jnp.float32),
            pltpu.VMEM((_CH, _D), jnp.float32),
            pltpu.VMEM((_CH,), jnp.int32),
            pltpu.VMEM((_CH,), jnp.int32),
            pltpu.VMEM((_CH,), jnp.int32),
            pltpu.VMEM((_CH,), jnp.int32),
            pltpu.VMEM((16,), jnp.int32),
            pltpu.VMEM((16,), jnp.int32),
        ] + [pltpu.SemaphoreType.DMA] * 9,
    )
    asp, alen = sc_call(ctx_embeddings, posp, zsrc)

    ctx, clen = pl.pallas_call(
        _tc_body,
        grid=(_B,),
        in_specs=[
            pl.BlockSpec((1, 1, _LRAW), lambda b: (b, 0, 0)),
            pl.BlockSpec((1, _LRAW, _D), lambda b: (b, 0, 0)),
        ],
        out_specs=[
            pl.BlockSpec((1, _L, _D), lambda b: (b, 0, 0)),
            pl.BlockSpec(memory_space=pltpu.SMEM),
        ],
        out_shape=[
            jax.ShapeDtypeStruct((_B, _L, _D), jnp.float32),
            jax.ShapeDtypeStruct((_B,), jnp.int32),
        ],
    )(mask3, ctx_embeddings)

    return (ctx, asp, clen, alen[:, 0])
